# Initial kernel scaffold; baseline (speedup 1.0000x reference)
import jax
import jax.numpy as jnp
from jax.experimental import pallas as pl


def kernel(z, w1, b1, gamma, beta, w2, b2):
    raise NotImplementedError("write your pallas kernel here")


# speedup vs baseline: 1.0980x; 1.0980x over previous
import functools

import jax
import jax.numpy as jnp
from jax.experimental import pallas as pl
from jax.experimental.pallas import tpu as pltpu

_BN_EPS = 1e-5
_INV_SQRT2 = 0.7071067811865476


def _round_up(x, m):
    return (x + m - 1) // m * m


def _gram_kernel(x_ref, gram_ref, csum_ref):
    i = pl.program_id(1)

    @pl.when(i == 0)
    def _():
        gram_ref[...] = jnp.zeros_like(gram_ref)
        csum_ref[...] = jnp.zeros_like(csum_ref)

    xb = x_ref[...]
    xb16 = xb.astype(jnp.bfloat16)
    g = jax.lax.dot_general(
        xb16, xb16,
        dimension_numbers=(((0,), (0,)), ((), ())),
        preferred_element_type=jnp.float32)
    gram_ref[...] += g[None]
    tb, d = xb.shape
    csum_ref[...] += jnp.sum(xb.reshape(tb // 8, 8, d), axis=0)[None]


def _affine_kernel(gram_ref, csum_ref, w1_ref, gamma_ref, beta_ref,
                   scale_ref, shift_ref, *, true_b):
    inv_b = 1.0 / true_b
    gram = jnp.sum(gram_ref[...], axis=0)
    cs = jnp.sum(csum_ref[...], axis=(0, 1), keepdims=False)
    cs = cs.reshape(1, -1)

    w1 = w1_ref[...]
    mean = jnp.dot(cs, w1, preferred_element_type=jnp.float32) * inv_b
    m = jnp.dot(gram, w1, preferred_element_type=jnp.float32)
    ex2 = jnp.sum(w1 * m, axis=0, keepdims=True) * inv_b
    var = jnp.maximum(ex2 - mean * mean, 0.0)
    scale = gamma_ref[...] * jax.lax.rsqrt(var + _BN_EPS)
    scale_ref[...] = scale
    shift_ref[...] = beta_ref[...] - mean * scale


def _apply_kernel(x_ref, w1_ref, scale_ref, shift_ref, w2_ref, b2_ref, o_ref):
    h0 = jnp.dot(x_ref[...].astype(jnp.bfloat16), w1_ref[...],
                 preferred_element_type=jnp.float32)
    hn = h0 * scale_ref[...] + shift_ref[...]
    g = 0.5 * hn * (1.0 + jax.lax.erf(hn * _INV_SQRT2))
    out = jnp.dot(g.astype(jnp.bfloat16), w2_ref[...],
                  preferred_element_type=jnp.float32) + b2_ref[...]
    o_ref[...] = out.astype(o_ref.dtype)


def kernel(z, w1, b1, gamma, beta, w2, b2, *, tile_b=1024):
    del b1
    B, in_dim = z.shape
    H = w1.shape[1]
    out_dim = w2.shape[1]
    f32 = jnp.float32

    tb = min(tile_b, max(8, _round_up(pl.cdiv(B, 4), 8)))
    b_p = _round_up(B, tb)
    n_tiles = b_p // tb

    z = z.astype(f32)
    if b_p != B:
        z = jnp.pad(z, ((0, b_p - B), (0, 0)))

    w1f = w1.astype(f32)
    w1_16 = w1.astype(jnp.bfloat16)
    w2_16 = w2.astype(jnp.bfloat16)
    gamma = gamma.astype(f32).reshape(1, H)
    beta = beta.astype(f32).reshape(1, H)
    b2 = b2.astype(f32).reshape(1, out_dim)

    n_split = 2 if (n_tiles >= 2 and n_tiles % 2 == 0) else 1
    tps = n_tiles // n_split

    gram, csum = pl.pallas_call(
        _gram_kernel,
        out_shape=(jax.ShapeDtypeStruct((n_split, in_dim, in_dim), f32),
                   jax.ShapeDtypeStruct((n_split, 8, in_dim), f32)),
        grid=(n_split, tps),
        in_specs=[pl.BlockSpec((tb, in_dim), lambda c, i: (c * tps + i, 0))],
        out_specs=(pl.BlockSpec((1, in_dim, in_dim), lambda c, i: (c, 0, 0)),
                   pl.BlockSpec((1, 8, in_dim), lambda c, i: (c, 0, 0))),
        compiler_params=pltpu.CompilerParams(
            dimension_semantics=("parallel", "arbitrary")),
    )(z)

    scale, shift = pl.pallas_call(
        functools.partial(_affine_kernel, true_b=B),
        out_shape=(jax.ShapeDtypeStruct((1, H), f32),
                   jax.ShapeDtypeStruct((1, H), f32)),
    )(gram, csum, w1f, gamma, beta)

    const2 = lambda i: (0, 0)
    out = pl.pallas_call(
        _apply_kernel,
        out_shape=jax.ShapeDtypeStruct((b_p, out_dim), f32),
        grid=(n_tiles,),
        in_specs=[pl.BlockSpec((tb, in_dim), lambda i: (i, 0)),
                  pl.BlockSpec((in_dim, H), const2),
                  pl.BlockSpec((1, H), const2),
                  pl.BlockSpec((1, H), const2),
                  pl.BlockSpec((H, out_dim), const2),
                  pl.BlockSpec((1, out_dim), const2)],
        out_specs=pl.BlockSpec((tb, out_dim), lambda i: (i, 0)),
        compiler_params=pltpu.CompilerParams(
            dimension_semantics=("parallel",)),
    )(z, w1_16, scale, shift, w2_16, b2)

    if b_p != B:
        out = out[:B]
    return out


# speedup vs baseline: 1.2382x; 1.1277x over previous
import functools
import math

import jax
import jax.numpy as jnp
from jax.experimental import pallas as pl
from jax.experimental.pallas import tpu as pltpu

_BN_EPS = 1e-5
_INV_SQRT2 = 0.7071067811865476


def _round_up(x, m):
    return (x + m - 1) // m * m


def _stats_kernel(x_ref, w1_ref, gamma_ref, beta_ref,
                  w1s_ref, shift2_ref, gram_ref, csum_ref, *,
                  n_steps, true_b):
    i = pl.program_id(0)

    @pl.when(i == 0)
    def _():
        gram_ref[...] = jnp.zeros_like(gram_ref)
        csum_ref[...] = jnp.zeros_like(csum_ref)

    xb = x_ref[...]
    xb16 = xb.astype(jnp.bfloat16)
    gram_ref[...] += jax.lax.dot_general(
        xb16, xb16,
        dimension_numbers=(((0,), (0,)), ((), ())),
        preferred_element_type=jnp.float32)
    tb, d = xb.shape
    csum_ref[...] += jnp.sum(xb.reshape(tb // 8, 8, d), axis=0)

    @pl.when(i == n_steps - 1)
    def _():
        inv_b = 1.0 / true_b
        w1 = w1_ref[...]
        cs = jnp.sum(csum_ref[...], axis=0, keepdims=True)
        mean = jnp.dot(cs, w1, preferred_element_type=jnp.float32) * inv_b
        m = jnp.dot(gram_ref[...], w1, preferred_element_type=jnp.float32)
        ex2 = jnp.sum(w1 * m, axis=0, keepdims=True) * inv_b
        var = jnp.maximum(ex2 - mean * mean, 0.0)
        scale = gamma_ref[...] * jax.lax.rsqrt(var + _BN_EPS)
        sc = scale * _INV_SQRT2
        w1s_ref[...] = (w1 * sc).astype(jnp.bfloat16)
        shift2_ref[...] = (beta_ref[...] - mean * scale) * _INV_SQRT2


def _apply_kernel(x_ref, w1s_ref, shift2_ref, w2s_ref, b2_ref, o_ref):
    u = jnp.dot(x_ref[...].astype(jnp.bfloat16), w1s_ref[...],
                preferred_element_type=jnp.float32) + shift2_ref[...]
    g = u + u * jax.lax.erf(u)
    out = jnp.dot(g.astype(jnp.bfloat16), w2s_ref[...],
                  preferred_element_type=jnp.float32) + b2_ref[...]
    o_ref[...] = out.astype(o_ref.dtype)


def kernel(z, w1, b1, gamma, beta, w2, b2, *, stats_tile_b=2048, tile_b=1024):
    del b1
    B, in_dim = z.shape
    H = w1.shape[1]
    out_dim = w2.shape[1]
    f32 = jnp.float32

    tb1 = min(stats_tile_b, max(8, _round_up(B, 8)))
    tb2 = min(tile_b, max(8, _round_up(B, 8)))
    b_p = _round_up(B, tb1 * tb2 // math.gcd(tb1, tb2))
    z = z.astype(f32)
    if b_p != B:
        z = jnp.pad(z, ((0, b_p - B), (0, 0)))
    n1 = b_p // tb1
    n2 = b_p // tb2

    w1f = w1.astype(f32)
    w2s = (w2.astype(f32) * _INV_SQRT2).astype(jnp.bfloat16)
    gamma = gamma.astype(f32).reshape(1, H)
    beta = beta.astype(f32).reshape(1, H)
    b2 = b2.astype(f32).reshape(1, out_dim)

    const1 = lambda i: (0, 0)
    w1s, shift2 = pl.pallas_call(
        functools.partial(_stats_kernel, n_steps=n1, true_b=B),
        out_shape=(jax.ShapeDtypeStruct((in_dim, H), jnp.bfloat16),
                   jax.ShapeDtypeStruct((1, H), f32)),
        grid=(n1,),
        in_specs=[pl.BlockSpec((tb1, in_dim), lambda i: (i, 0)),
                  pl.BlockSpec((in_dim, H), const1),
                  pl.BlockSpec((1, H), const1),
                  pl.BlockSpec((1, H), const1)],
        out_specs=(pl.BlockSpec((in_dim, H), const1),
                   pl.BlockSpec((1, H), const1)),
        scratch_shapes=[pltpu.VMEM((in_dim, in_dim), f32),
                        pltpu.VMEM((8, in_dim), f32)],
        compiler_params=pltpu.CompilerParams(
            dimension_semantics=("arbitrary",)),
    )(z, w1f, gamma, beta)

    const2 = lambda i: (0, 0)
    out = pl.pallas_call(
        _apply_kernel,
        out_shape=jax.ShapeDtypeStruct((b_p, out_dim), f32),
        grid=(n2,),
        in_specs=[pl.BlockSpec((tb2, in_dim), lambda i: (i, 0)),
                  pl.BlockSpec((in_dim, H), const2),
                  pl.BlockSpec((1, H), const2),
                  pl.BlockSpec((H, out_dim), const2),
                  pl.BlockSpec((1, out_dim), const2)],
        out_specs=pl.BlockSpec((tb2, out_dim), lambda i: (i, 0)),
        compiler_params=pltpu.CompilerParams(
            dimension_semantics=("arbitrary",)),
    )(z, w1s, shift2, w2s, b2)

    if b_p != B:
        out = out[:B]
    return out
